# baseline (speedup 1.0000x reference)
import functools

import jax
import jax.numpy as jnp
from jax import lax
from jax.experimental import pallas as pl
from jax.experimental.pallas import tpu as pltpu
from jax.experimental.pallas import tpu_sc as plsc

N_LAYERS = 3
N_NODES = 10000
N_PAD = 10240
N_EDGES = 320000
D = 128
DH = D // 2

NC = 2
NS = 16
L = 16

GRP = 128
EDGES_PAD = 327680
NGRP = EDGES_PAD // GRP
GRP_PER_TILE = NGRP // NS
CHUNK_G = 4
CH = CHUNK_G * GRP
N_CHUNKS = GRP_PER_TILE // CHUNK_G

ROWS_PER_TILE = N_PAD // NS
RB = 128


def _body(t0_hbm, src_hbm, dst_hbm, w_hbm, out_hbm, tabs_hbm,
          rows_v, scaled_v, src_v, dst_v, w_v, zb_v, nxt_sp, sem):
    cid = lax.axis_index("c")
    sid = lax.axis_index("s")
    r0 = sid * ROWS_PER_TILE

    lane = lax.iota(jnp.int32, L)

    def zrow(i, _):
        for cseg in range(DH // L):
            zb_v[i, pl.ds(cseg * L, L)] = jnp.zeros((L,), jnp.float32)
        return 0
    lax.fori_loop(0, RB, zrow, 0)

    def run_layer(in_hbm, nxt_sp):
        for b in range(ROWS_PER_TILE // RB):
            pltpu.sync_copy(zb_v, nxt_sp.at[pl.ds(r0 + b * RB, RB)])
        plsc.subcore_barrier()

        def chunk_body(ci, _):
            g0 = sid * GRP_PER_TILE + ci * CHUNK_G
            pltpu.sync_copy(src_hbm.at[pl.ds(g0, CHUNK_G)], src_v)
            pltpu.sync_copy(dst_hbm.at[pl.ds(g0, CHUNK_G)], dst_v)
            pltpu.sync_copy(w_hbm.at[pl.ds(g0, CHUNK_G)], w_v)

            cps = []
            for g in range(CHUNK_G):
                cps.append(pltpu.async_copy(
                    in_hbm.at[src_v.at[g]],
                    rows_v.at[pl.ds(g * GRP, GRP)], sem))
            for cp in cps:
                cp.wait()

            @plsc.parallel_loop(0, GRP // L, unroll=2)
            def scale_grp(k):
                for g in range(CHUNK_G):
                    wv = w_v[g, pl.ds(k * L, L)]
                    for e in range(L):
                        r = g * GRP + k * L + e
                        w = wv[e]
                        for s in range(DH // L):
                            sl = pl.ds(s * L, L)
                            scaled_v[r, sl] = rows_v[r, sl] * w

            for g in range(CHUNK_G):
                pltpu.sync_copy(scaled_v.at[pl.ds(g * GRP, GRP)],
                                nxt_sp.at[dst_v.at[g]], add=True)
            return 0
        lax.fori_loop(0, N_CHUNKS, chunk_body, 0)
        plsc.subcore_barrier()

    def spmem_to_hbm(sp, hbm_slice):
        stg = rows_v.at[pl.ds(0, ROWS_PER_TILE)]
        pltpu.sync_copy(sp.at[pl.ds(r0, ROWS_PER_TILE)], stg)
        pltpu.sync_copy(stg, hbm_slice.at[pl.ds(r0, ROWS_PER_TILE)])

    if True:
        for l in range(N_LAYERS):
            in_hbm = t0_hbm.at[cid] if l == 0 else tabs_hbm.at[l - 1, cid]
            run_layer(in_hbm, nxt_sp)
            spmem_to_hbm(nxt_sp, tabs_hbm.at[l, cid])
            plsc.subcore_barrier()

        def addrow(i, _):
            for cseg in range(DH // L):
                sl = pl.ds(cseg * L, L)
                zb_v[i, sl] = zb_v[i, sl] + rows_v[i, sl]
            return 0

        def mulrow(i, _):
            for cseg in range(DH // L):
                sl = pl.ds(cseg * L, L)
                zb_v[i, sl] = zb_v[i, sl] * 0.25
            return 0

        for b in range(ROWS_PER_TILE // RB):
            rblk = pl.ds(r0 + b * RB, RB)
            pltpu.sync_copy(t0_hbm.at[cid].at[rblk], zb_v)
            for l in range(N_LAYERS):
                pltpu.sync_copy(tabs_hbm.at[l, cid].at[rblk],
                                rows_v.at[pl.ds(0, RB)])
                lax.fori_loop(0, RB, addrow, 0)
            lax.fori_loop(0, RB, mulrow, 0)
            pltpu.sync_copy(zb_v, out_hbm.at[cid].at[rblk])


@jax.jit
def kernel(users_emb, items_emb, edge_index, edge_weight):
    num_users = users_emb.shape[0]
    all_emb = jnp.concatenate(
        [users_emb, items_emb,
         jnp.zeros((N_PAD - N_NODES, D), jnp.float32)], axis=0)
    t0 = all_emb.reshape(N_PAD, NC, DH).transpose(1, 0, 2)

    dst = edge_index[0].astype(jnp.int32)
    src = edge_index[1].astype(jnp.int32)
    w = edge_weight

    npad = EDGES_PAD - N_EDGES
    pad_idx = (jnp.arange(npad, dtype=jnp.int32) % N_NODES)
    srcp = jnp.concatenate([src, pad_idx]).reshape(NGRP, GRP)
    dstp = jnp.concatenate([dst, pad_idx]).reshape(NGRP, GRP)
    wp = jnp.concatenate([w, jnp.zeros((npad,), jnp.float32)]).reshape(NGRP, GRP)

    mesh = plsc.VectorSubcoreMesh(core_axis_name="c", subcore_axis_name="s")
    out, _tabs = pl.kernel(
        _body,
        out_type=(
            jax.ShapeDtypeStruct((NC, N_PAD, DH), jnp.float32),
            jax.ShapeDtypeStruct((N_LAYERS, NC, N_PAD, DH), jnp.float32),
        ),
        mesh=mesh,
        compiler_params=pltpu.CompilerParams(
            needs_layout_passes=False, use_tc_tiling_on_sc=False),
        scratch_types=[
            pltpu.VMEM((ROWS_PER_TILE, DH), jnp.float32),
            pltpu.VMEM((CH, DH), jnp.float32),
            pltpu.VMEM((CHUNK_G, GRP), jnp.int32),
            pltpu.VMEM((CHUNK_G, GRP), jnp.int32),
            pltpu.VMEM((CHUNK_G, GRP), jnp.float32),
            pltpu.VMEM((RB, DH), jnp.float32),
            pltpu.VMEM_SHARED((N_PAD, DH), jnp.float32),
            pltpu.SemaphoreType.DMA,
        ],
    )(t0, srcp, dstp, wp)

    light = out.transpose(1, 0, 2).reshape(N_PAD, D)[:N_NODES]
    return (light[:num_users], light[num_users:])


# speedup vs baseline: 2.2782x; 2.2782x over previous
import functools

import jax
import jax.numpy as jnp
from jax import lax
from jax.experimental import pallas as pl
from jax.experimental.pallas import tpu as pltpu
from jax.experimental.pallas import tpu_sc as plsc

N_LAYERS = 3
N_NODES = 10000
N_PAD = 10240
N_EDGES = 320000
D = 128
DH = D // 2

NC = 2
NS = 16
L = 16

GRP = 128
EDGES_PAD = 327680
NGRP = EDGES_PAD // GRP
GRP_PER_TILE = NGRP // NS
CHUNK_G = 4
CH = CHUNK_G * GRP
N_CHUNKS = GRP_PER_TILE // CHUNK_G

ROWS_PER_TILE = N_PAD // NS
RB = 128


def _body(t0_hbm, src_hbm, dst_hbm, w_hbm, out_hbm, tabs_hbm,
          rows_v, scaled_v, src_v, dst_v, w_v, zb_v, nxt_sp, sem):
    cid = lax.axis_index("c")
    sid = lax.axis_index("s")
    r0 = sid * ROWS_PER_TILE

    lane = lax.iota(jnp.int32, L)

    def zrow(i, _):
        for cseg in range(DH // L):
            zb_v[i, pl.ds(cseg * L, L)] = jnp.zeros((L,), jnp.float32)
        return 0
    lax.fori_loop(0, RB, zrow, 0)

    def run_layer(in_hbm, nxt_sp):
        for b in range(ROWS_PER_TILE // RB):
            pltpu.sync_copy(zb_v, nxt_sp.at[pl.ds(r0 + b * RB, RB)])
        plsc.subcore_barrier()

        def chunk_body(ci, _):
            g0 = sid * GRP_PER_TILE + ci * CHUNK_G
            pltpu.sync_copy(src_hbm.at[pl.ds(g0, CHUNK_G)], src_v)
            pltpu.sync_copy(dst_hbm.at[pl.ds(g0, CHUNK_G)], dst_v)
            pltpu.sync_copy(w_hbm.at[pl.ds(g0, CHUNK_G)], w_v)

            cps = []
            for g in range(CHUNK_G):
                cps.append(pltpu.async_copy(
                    in_hbm.at[src_v.at[g]],
                    rows_v.at[pl.ds(g * GRP, GRP)], sem))
            for cp in cps:
                cp.wait()

            def scale_grp(k, _):
                for g in range(CHUNK_G):
                    wv = w_v[g, pl.ds(k * L, L)]
                    for e in range(L):
                        r = g * GRP + k * L + e
                        w = wv[e]
                        for s in range(DH // L):
                            sl = pl.ds(s * L, L)
                            rows_v[r, sl] = rows_v[r, sl] * w
                return 0
            lax.fori_loop(0, GRP // L, scale_grp, 0)

            for g in range(CHUNK_G):
                pltpu.sync_copy(rows_v.at[pl.ds(g * GRP, GRP)],
                                nxt_sp.at[dst_v.at[g]], add=True)
            return 0
        lax.fori_loop(0, N_CHUNKS, chunk_body, 0)
        plsc.subcore_barrier()

    def spmem_to_hbm(sp, hbm_slice):
        stg = rows_v.at[pl.ds(0, ROWS_PER_TILE)]
        pltpu.sync_copy(sp.at[pl.ds(r0, ROWS_PER_TILE)], stg)
        pltpu.sync_copy(stg, hbm_slice.at[pl.ds(r0, ROWS_PER_TILE)])

    if True:
        for l in range(N_LAYERS):
            in_hbm = t0_hbm.at[cid] if l == 0 else tabs_hbm.at[l - 1, cid]
            run_layer(in_hbm, nxt_sp)
            spmem_to_hbm(nxt_sp, tabs_hbm.at[l, cid])
            plsc.subcore_barrier()

        def addrow(i, _):
            for cseg in range(DH // L):
                sl = pl.ds(cseg * L, L)
                zb_v[i, sl] = zb_v[i, sl] + rows_v[i, sl]
            return 0

        def mulrow(i, _):
            for cseg in range(DH // L):
                sl = pl.ds(cseg * L, L)
                zb_v[i, sl] = zb_v[i, sl] * 0.25
            return 0

        for b in range(ROWS_PER_TILE // RB):
            rblk = pl.ds(r0 + b * RB, RB)
            pltpu.sync_copy(t0_hbm.at[cid].at[rblk], zb_v)
            for l in range(N_LAYERS):
                pltpu.sync_copy(tabs_hbm.at[l, cid].at[rblk],
                                rows_v.at[pl.ds(0, RB)])
                lax.fori_loop(0, RB, addrow, 0)
            lax.fori_loop(0, RB, mulrow, 0)
            pltpu.sync_copy(zb_v, out_hbm.at[cid].at[rblk])


@jax.jit
def kernel(users_emb, items_emb, edge_index, edge_weight):
    num_users = users_emb.shape[0]
    all_emb = jnp.concatenate(
        [users_emb, items_emb,
         jnp.zeros((N_PAD - N_NODES, D), jnp.float32)], axis=0)
    t0 = all_emb.reshape(N_PAD, NC, DH).transpose(1, 0, 2)

    dst = edge_index[0].astype(jnp.int32)
    src = edge_index[1].astype(jnp.int32)
    w = edge_weight

    npad = EDGES_PAD - N_EDGES
    pad_idx = (jnp.arange(npad, dtype=jnp.int32) % N_NODES)
    srcp = jnp.concatenate([src, pad_idx]).reshape(NGRP, GRP)
    dstp = jnp.concatenate([dst, pad_idx]).reshape(NGRP, GRP)
    wp = jnp.concatenate([w, jnp.zeros((npad,), jnp.float32)]).reshape(NGRP, GRP)

    mesh = plsc.VectorSubcoreMesh(core_axis_name="c", subcore_axis_name="s")
    out, _tabs = pl.kernel(
        _body,
        out_type=(
            jax.ShapeDtypeStruct((NC, N_PAD, DH), jnp.float32),
            jax.ShapeDtypeStruct((N_LAYERS, NC, N_PAD, DH), jnp.float32),
        ),
        mesh=mesh,
        compiler_params=pltpu.CompilerParams(
            needs_layout_passes=False, use_tc_tiling_on_sc=False),
        scratch_types=[
            pltpu.VMEM((ROWS_PER_TILE, DH), jnp.float32),
            pltpu.VMEM((CH, DH), jnp.float32),
            pltpu.VMEM((CHUNK_G, GRP), jnp.int32),
            pltpu.VMEM((CHUNK_G, GRP), jnp.int32),
            pltpu.VMEM((CHUNK_G, GRP), jnp.float32),
            pltpu.VMEM((RB, DH), jnp.float32),
            pltpu.VMEM_SHARED((N_PAD, DH), jnp.float32),
            pltpu.SemaphoreType.DMA,
        ],
    )(t0, srcp, dstp, wp)

    light = out.transpose(1, 0, 2).reshape(N_PAD, D)[:N_NODES]
    return (light[:num_users], light[num_users:])


# speedup vs baseline: 3.2984x; 1.4478x over previous
import functools

import jax
import jax.numpy as jnp
from jax import lax
from jax.experimental import pallas as pl
from jax.experimental.pallas import tpu as pltpu
from jax.experimental.pallas import tpu_sc as plsc

N_LAYERS = 3
N_NODES = 10000
N_PAD = 10240
N_EDGES = 320000
D = 128
DH = D // 2

NC = 2
NS = 16
L = 16

GRP = 128
EDGES_PAD = 327680
NGRP = EDGES_PAD // GRP
GRP_PER_TILE = NGRP // NS
CHUNK_G = 2
CH = CHUNK_G * GRP
N_CHUNKS = GRP_PER_TILE // CHUNK_G
NBUF = 3
N_TRIPLES = (N_CHUNKS - 2) // NBUF

ROWS_PER_TILE = N_PAD // NS
RB = 128


def _body(t0_hbm, idx_hbm, out_hbm, tabs_hbm,
          rows_v, idx_v, zb_v, nxt_sp,
          gs0, gs1, gs2, as0, as1, as2):
    cid = lax.axis_index("c")
    sid = lax.axis_index("s")
    r0 = sid * ROWS_PER_TILE
    g_base = sid * GRP_PER_TILE
    gsems = [gs0, gs1, gs2]
    asems = [as0, as1, as2]

    def zrow(i, _):
        for cseg in range(DH // L):
            zb_v[i, pl.ds(cseg * L, L)] = jnp.zeros((L,), jnp.float32)
        return 0
    lax.fori_loop(0, RB, zrow, 0)

    def load_idx(ci, b):
        g0 = g_base + ci * CHUNK_G
        pltpu.sync_copy(idx_hbm.at[pl.ds(g0, CHUNK_G)], idx_v.at[b])

    def issue_gathers(in_hbm, b):
        for g in range(CHUNK_G):
            pltpu.async_copy(in_hbm.at[idx_v.at[b].at[g].at[0]],
                             rows_v.at[b].at[pl.ds(g * GRP, GRP)], gsems[b])

    def drain_gathers(in_hbm, b):
        for g in range(CHUNK_G):
            pltpu.make_async_copy(
                in_hbm.at[idx_v.at[b].at[g].at[0]],
                rows_v.at[b].at[pl.ds(g * GRP, GRP)], gsems[b]).wait()

    def issue_adds(nxt_sp, b):
        for g in range(CHUNK_G):
            pltpu.async_copy(rows_v.at[b].at[pl.ds(g * GRP, GRP)],
                             nxt_sp.at[idx_v.at[b].at[g].at[1]],
                             asems[b], add=True)

    def drain_adds(nxt_sp, b):
        for g in range(CHUNK_G):
            pltpu.make_async_copy(
                rows_v.at[b].at[pl.ds(g * GRP, GRP)],
                nxt_sp.at[idx_v.at[b].at[g].at[1]], asems[b]).wait()

    def scale(b):
        def scale_grp(k, _):
            for g in range(CHUNK_G):
                wv = plsc.bitcast(idx_v[b, g, 2, pl.ds(k * L, L)],
                                  jnp.float32)
                for e in range(L):
                    r = g * GRP + k * L + e
                    w = wv[e]
                    for s in range(DH // L):
                        sl = pl.ds(s * L, L)
                        rows_v[b, r, sl] = rows_v[b, r, sl] * w
            return 0
        lax.fori_loop(0, GRP // L, scale_grp, 0)

    def run_layer(in_hbm, nxt_sp):
        for b in range(ROWS_PER_TILE // RB):
            pltpu.sync_copy(zb_v, nxt_sp.at[pl.ds(r0 + b * RB, RB)])
        plsc.subcore_barrier()

        load_idx(0, 0)
        issue_gathers(in_hbm, 0)

        def pipe(ii, _):
            for j in range(NBUF):
                c = ii * NBUF + j
                j1 = (j + 1) % NBUF
                drain_gathers(in_hbm, j)

                @pl.when(c >= 2)
                def _():
                    drain_adds(nxt_sp, j1)
                load_idx(c + 1, j1)
                issue_gathers(in_hbm, j1)
                scale(j)
                issue_adds(nxt_sp, j)
            return 0
        lax.fori_loop(0, N_TRIPLES, pipe, 0)

        drain_gathers(in_hbm, 0)
        drain_adds(nxt_sp, 1)
        load_idx(N_CHUNKS - 1, 1)
        issue_gathers(in_hbm, 1)
        scale(0)
        issue_adds(nxt_sp, 0)

        drain_gathers(in_hbm, 1)
        scale(1)
        issue_adds(nxt_sp, 1)
        drain_adds(nxt_sp, 2)
        drain_adds(nxt_sp, 0)
        drain_adds(nxt_sp, 1)
        plsc.subcore_barrier()

    def spmem_to_hbm(sp, hbm_slice):
        pltpu.sync_copy(sp.at[pl.ds(r0, CH)], rows_v.at[0])
        pltpu.sync_copy(sp.at[pl.ds(r0 + CH, CH)], rows_v.at[1])
        pltpu.sync_copy(sp.at[pl.ds(r0 + 2 * CH, RB)],
                        rows_v.at[2].at[pl.ds(0, RB)])
        pltpu.sync_copy(rows_v.at[0], hbm_slice.at[pl.ds(r0, CH)])
        pltpu.sync_copy(rows_v.at[1], hbm_slice.at[pl.ds(r0 + CH, CH)])
        pltpu.sync_copy(rows_v.at[2].at[pl.ds(0, RB)],
                        hbm_slice.at[pl.ds(r0 + 2 * CH, RB)])

    for l in range(N_LAYERS):
        in_hbm = t0_hbm.at[cid] if l == 0 else tabs_hbm.at[l - 1, cid]
        run_layer(in_hbm, nxt_sp)
        spmem_to_hbm(nxt_sp, tabs_hbm.at[l, cid])
        plsc.subcore_barrier()

    def addrow(i, _):
        for cseg in range(DH // L):
            sl = pl.ds(cseg * L, L)
            zb_v[i, sl] = zb_v[i, sl] + rows_v[0, i, sl]
        return 0

    def mulrow(i, _):
        for cseg in range(DH // L):
            sl = pl.ds(cseg * L, L)
            zb_v[i, sl] = zb_v[i, sl] * 0.25
        return 0

    for b in range(ROWS_PER_TILE // RB):
        rblk = pl.ds(r0 + b * RB, RB)
        pltpu.sync_copy(t0_hbm.at[cid].at[rblk], zb_v)
        for l in range(N_LAYERS):
            pltpu.sync_copy(tabs_hbm.at[l, cid].at[rblk],
                            rows_v.at[0].at[pl.ds(0, RB)])
            lax.fori_loop(0, RB, addrow, 0)
        lax.fori_loop(0, RB, mulrow, 0)
        pltpu.sync_copy(zb_v, out_hbm.at[cid].at[rblk])


@jax.jit
def kernel(users_emb, items_emb, edge_index, edge_weight):
    num_users = users_emb.shape[0]
    all_emb = jnp.concatenate(
        [users_emb, items_emb,
         jnp.zeros((N_PAD - N_NODES, D), jnp.float32)], axis=0)
    t0 = all_emb.reshape(N_PAD, NC, DH).transpose(1, 0, 2)

    dst = edge_index[0].astype(jnp.int32)
    src = edge_index[1].astype(jnp.int32)
    w = edge_weight

    npad = EDGES_PAD - N_EDGES
    pad_idx = (jnp.arange(npad, dtype=jnp.int32) % N_NODES)
    srcp = jnp.concatenate([src, pad_idx]).reshape(NGRP, GRP)
    dstp = jnp.concatenate([dst, pad_idx]).reshape(NGRP, GRP)
    wp = lax.bitcast_convert_type(
        jnp.concatenate([w, jnp.zeros((npad,), jnp.float32)]),
        jnp.int32).reshape(NGRP, GRP)
    idxp = jnp.stack([srcp, dstp, wp], axis=1)

    mesh = plsc.VectorSubcoreMesh(core_axis_name="c", subcore_axis_name="s")
    out, _tabs = pl.kernel(
        _body,
        out_type=(
            jax.ShapeDtypeStruct((NC, N_PAD, DH), jnp.float32),
            jax.ShapeDtypeStruct((N_LAYERS, NC, N_PAD, DH), jnp.float32),
        ),
        mesh=mesh,
        compiler_params=pltpu.CompilerParams(
            needs_layout_passes=False, use_tc_tiling_on_sc=False),
        scratch_types=[
            pltpu.VMEM((NBUF, CH, DH), jnp.float32),
            pltpu.VMEM((NBUF, CHUNK_G, 3, GRP), jnp.int32),
            pltpu.VMEM((RB, DH), jnp.float32),
            pltpu.VMEM_SHARED((N_PAD, DH), jnp.float32),
            pltpu.SemaphoreType.DMA,
            pltpu.SemaphoreType.DMA,
            pltpu.SemaphoreType.DMA,
            pltpu.SemaphoreType.DMA,
            pltpu.SemaphoreType.DMA,
            pltpu.SemaphoreType.DMA,
        ],
    )(t0, idxp)

    light = out.transpose(1, 0, 2).reshape(N_PAD, D)[:N_NODES]
    return (light[:num_users], light[num_users:])


# speedup vs baseline: 3.9739x; 1.2048x over previous
import functools

import jax
import jax.numpy as jnp
from jax import lax
from jax.experimental import pallas as pl
from jax.experimental.pallas import tpu as pltpu
from jax.experimental.pallas import tpu_sc as plsc

N_LAYERS = 3
N_NODES = 10000
N_PAD = 10240
N_EDGES = 320000
D = 128
DH = D // 2

NC = 2
NS = 16
L = 16

GRP = 128
EDGES_PAD = 327680
NGRP = EDGES_PAD // GRP
GRP_PER_TILE = NGRP // NS
CHUNK_G = 2
CH = CHUNK_G * GRP
N_CHUNKS = GRP_PER_TILE // CHUNK_G
NBUF = 4

ROWS_PER_TILE = N_PAD // NS
RB = 128


def _body(t0_hbm, idx_hbm, out_hbm, tabs_hbm,
          rows_v, idx_v, zb_v, nxt_sp,
          gs0, gs1, gs2, gs3, as0, as1, as2, as3, is0, is1, is2, is3):
    cid = lax.axis_index("c")
    sid = lax.axis_index("s")
    r0 = sid * ROWS_PER_TILE
    g_base = sid * GRP_PER_TILE
    gsems = [gs0, gs1, gs2, gs3]
    asems = [as0, as1, as2, as3]
    isems = [is0, is1, is2, is3]

    def zrow(i, _):
        for cseg in range(DH // L):
            zb_v[i, pl.ds(cseg * L, L)] = jnp.zeros((L,), jnp.float32)
        return 0
    lax.fori_loop(0, RB, zrow, 0)

    def load_idx_async(ci, b):
        g0 = g_base + ci * CHUNK_G
        pltpu.async_copy(idx_hbm.at[pl.ds(g0, CHUNK_G)], idx_v.at[b],
                         isems[b])

    def wait_idx(ci, b):
        g0 = g_base + ci * CHUNK_G
        pltpu.make_async_copy(idx_hbm.at[pl.ds(g0, CHUNK_G)], idx_v.at[b],
                              isems[b]).wait()

    def issue_gathers(in_hbm, b):
        for g in range(CHUNK_G):
            pltpu.async_copy(in_hbm.at[idx_v.at[b].at[g].at[0]],
                             rows_v.at[b].at[pl.ds(g * GRP, GRP)], gsems[b])

    def drain_gathers(in_hbm, b):
        for g in range(CHUNK_G):
            pltpu.make_async_copy(
                in_hbm.at[idx_v.at[b].at[g].at[0]],
                rows_v.at[b].at[pl.ds(g * GRP, GRP)], gsems[b]).wait()

    def issue_adds(nxt_sp, b):
        for g in range(CHUNK_G):
            pltpu.async_copy(rows_v.at[b].at[pl.ds(g * GRP, GRP)],
                             nxt_sp.at[idx_v.at[b].at[g].at[1]],
                             asems[b], add=True)

    def drain_adds(nxt_sp, b):
        for g in range(CHUNK_G):
            pltpu.make_async_copy(
                rows_v.at[b].at[pl.ds(g * GRP, GRP)],
                nxt_sp.at[idx_v.at[b].at[g].at[1]], asems[b]).wait()

    def scale(b):
        def scale_grp(k, _):
            for g in range(CHUNK_G):
                wv = plsc.bitcast(idx_v[b, g, 2, pl.ds(k * L, L)],
                                  jnp.float32)
                for e in range(L):
                    r = g * GRP + k * L + e
                    w = wv[e]
                    for s in range(DH // L):
                        sl = pl.ds(s * L, L)
                        rows_v[b, r, sl] = rows_v[b, r, sl] * w
            return 0
        lax.fori_loop(0, GRP // L, scale_grp, 0)

    def run_layer(in_hbm, nxt_sp):
        for b in range(ROWS_PER_TILE // RB):
            pltpu.sync_copy(zb_v, nxt_sp.at[pl.ds(r0 + b * RB, RB)])
        plsc.subcore_barrier()

        pltpu.sync_copy(idx_hbm.at[pl.ds(g_base, CHUNK_G)], idx_v.at[0])
        issue_gathers(in_hbm, 0)
        load_idx_async(1, 1)

        def pipe(ii, _):
            for j in range(NBUF):
                c = ii * NBUF + j
                j1 = (j + 1) % NBUF
                j2 = (j + 2) % NBUF
                drain_gathers(in_hbm, j)

                @pl.when(c >= 2)
                def _():
                    drain_adds(nxt_sp, j2)

                @pl.when(c <= N_CHUNKS - 2)
                def _():
                    wait_idx(c + 1, j1)
                    issue_gathers(in_hbm, j1)

                @pl.when(c <= N_CHUNKS - 3)
                def _():
                    load_idx_async(c + 2, j2)
                scale(j)
                issue_adds(nxt_sp, j)
            return 0
        lax.fori_loop(0, N_CHUNKS // NBUF, pipe, 0)

        drain_adds(nxt_sp, (N_CHUNKS - 2) % NBUF)
        drain_adds(nxt_sp, (N_CHUNKS - 1) % NBUF)
        plsc.subcore_barrier()

    def spmem_to_hbm(sp, hbm_slice):
        pltpu.sync_copy(sp.at[pl.ds(r0, CH)], rows_v.at[0])
        pltpu.sync_copy(sp.at[pl.ds(r0 + CH, CH)], rows_v.at[1])
        pltpu.sync_copy(sp.at[pl.ds(r0 + 2 * CH, RB)],
                        rows_v.at[2].at[pl.ds(0, RB)])
        pltpu.sync_copy(rows_v.at[0], hbm_slice.at[pl.ds(r0, CH)])
        pltpu.sync_copy(rows_v.at[1], hbm_slice.at[pl.ds(r0 + CH, CH)])
        pltpu.sync_copy(rows_v.at[2].at[pl.ds(0, RB)],
                        hbm_slice.at[pl.ds(r0 + 2 * CH, RB)])

    for l in range(N_LAYERS):
        in_hbm = t0_hbm.at[cid] if l == 0 else tabs_hbm.at[l - 1, cid]
        run_layer(in_hbm, nxt_sp)
        spmem_to_hbm(nxt_sp, tabs_hbm.at[l, cid])
        plsc.subcore_barrier()

    def addrow(i, _):
        for cseg in range(DH // L):
            sl = pl.ds(cseg * L, L)
            zb_v[i, sl] = zb_v[i, sl] + rows_v[0, i, sl]
        return 0

    def mulrow(i, _):
        for cseg in range(DH // L):
            sl = pl.ds(cseg * L, L)
            zb_v[i, sl] = zb_v[i, sl] * 0.25
        return 0

    if True:
        for b in range(ROWS_PER_TILE // RB):
            rblk = pl.ds(r0 + b * RB, RB)
            pltpu.sync_copy(t0_hbm.at[cid].at[rblk], zb_v)
            for l in range(N_LAYERS):
                pltpu.sync_copy(tabs_hbm.at[l, cid].at[rblk],
                                rows_v.at[0].at[pl.ds(0, RB)])
                lax.fori_loop(0, RB, addrow, 0)
            lax.fori_loop(0, RB, mulrow, 0)
            pltpu.sync_copy(zb_v, out_hbm.at[cid].at[rblk])


@jax.jit
def kernel(users_emb, items_emb, edge_index, edge_weight):
    num_users = users_emb.shape[0]
    all_emb = jnp.concatenate(
        [users_emb, items_emb,
         jnp.zeros((N_PAD - N_NODES, D), jnp.float32)], axis=0)
    t0 = all_emb.reshape(N_PAD, NC, DH).transpose(1, 0, 2)

    dst = edge_index[0].astype(jnp.int32)
    src = edge_index[1].astype(jnp.int32)
    w = edge_weight

    npad = EDGES_PAD - N_EDGES
    pad_idx = (jnp.arange(npad, dtype=jnp.int32) % N_NODES)
    srcp = jnp.concatenate([src, pad_idx]).reshape(NGRP, GRP)
    dstp = jnp.concatenate([dst, pad_idx]).reshape(NGRP, GRP)
    wp = lax.bitcast_convert_type(
        jnp.concatenate([w, jnp.zeros((npad,), jnp.float32)]),
        jnp.int32).reshape(NGRP, GRP)
    idxp = jnp.stack([srcp, dstp, wp], axis=1)

    mesh = plsc.VectorSubcoreMesh(core_axis_name="c", subcore_axis_name="s")
    out, _tabs = pl.kernel(
        _body,
        out_type=(
            jax.ShapeDtypeStruct((NC, N_PAD, DH), jnp.float32),
            jax.ShapeDtypeStruct((N_LAYERS, NC, N_PAD, DH), jnp.float32),
        ),
        mesh=mesh,
        compiler_params=pltpu.CompilerParams(
            needs_layout_passes=False, use_tc_tiling_on_sc=False),
        scratch_types=[
            pltpu.VMEM((NBUF, CH, DH), jnp.float32),
            pltpu.VMEM((NBUF, CHUNK_G, 3, GRP), jnp.int32),
            pltpu.VMEM((RB, DH), jnp.float32),
            pltpu.VMEM_SHARED((N_PAD, DH), jnp.float32),
            pltpu.SemaphoreType.DMA,
            pltpu.SemaphoreType.DMA,
            pltpu.SemaphoreType.DMA,
            pltpu.SemaphoreType.DMA,
            pltpu.SemaphoreType.DMA,
            pltpu.SemaphoreType.DMA,
            pltpu.SemaphoreType.DMA,
            pltpu.SemaphoreType.DMA,
            pltpu.SemaphoreType.DMA,
            pltpu.SemaphoreType.DMA,
            pltpu.SemaphoreType.DMA,
            pltpu.SemaphoreType.DMA,
        ],
    )(t0, idxp)

    light = out.transpose(1, 0, 2).reshape(N_PAD, D)[:N_NODES]
    return (light[:num_users], light[num_users:])


# speedup vs baseline: 3.9951x; 1.0053x over previous
import functools

import jax
import jax.numpy as jnp
from jax import lax
from jax.experimental import pallas as pl
from jax.experimental.pallas import tpu as pltpu
from jax.experimental.pallas import tpu_sc as plsc

N_LAYERS = 3
N_NODES = 10000
N_PAD = 10240
N_EDGES = 320000
D = 128
DH = D // 2

NC = 2
NS = 16
L = 16

GRP = 128
EDGES_PAD = 327680
NGRP = EDGES_PAD // GRP
GRP_PER_TILE = NGRP // NS
CHUNK_G = 2
CH = CHUNK_G * GRP
N_CHUNKS = GRP_PER_TILE // CHUNK_G
NBUF = 4

ROWS_PER_TILE = N_PAD // NS
RB = 128


def _body(t0_hbm, idx_hbm, out_hbm, tabs_hbm,
          rows_v, idx_v, zb_v, nxt_sp,
          gs0, gs1, gs2, gs3, as0, as1, as2, as3, is0, is1, is2, is3):
    cid = lax.axis_index("c")
    sid = lax.axis_index("s")
    r0 = sid * ROWS_PER_TILE
    g_base = sid * GRP_PER_TILE
    gsems = [gs0, gs1, gs2, gs3]
    asems = [as0, as1, as2, as3]
    isems = [is0, is1, is2, is3]

    def zrow(i, _):
        for cseg in range(DH // L):
            zb_v[i, pl.ds(cseg * L, L)] = jnp.zeros((L,), jnp.float32)
        return 0
    lax.fori_loop(0, RB, zrow, 0)

    def load_idx_async(ci, b):
        g0 = g_base + ci * CHUNK_G
        pltpu.async_copy(idx_hbm.at[pl.ds(g0, CHUNK_G)], idx_v.at[b],
                         isems[b])

    def wait_idx(ci, b):
        g0 = g_base + ci * CHUNK_G
        pltpu.make_async_copy(idx_hbm.at[pl.ds(g0, CHUNK_G)], idx_v.at[b],
                              isems[b]).wait()

    def issue_gathers(in_hbm, b):
        for g in range(CHUNK_G):
            pltpu.async_copy(in_hbm.at[idx_v.at[b].at[g].at[0]],
                             rows_v.at[b].at[pl.ds(g * GRP, GRP)], gsems[b])

    def drain_gathers(in_hbm, b):
        for g in range(CHUNK_G):
            pltpu.make_async_copy(
                in_hbm.at[idx_v.at[b].at[g].at[0]],
                rows_v.at[b].at[pl.ds(g * GRP, GRP)], gsems[b]).wait()

    def issue_adds(nxt_sp, b):
        for g in range(CHUNK_G):
            pltpu.async_copy(rows_v.at[b].at[pl.ds(g * GRP, GRP)],
                             nxt_sp.at[idx_v.at[b].at[g].at[1]],
                             asems[b], add=True)

    def drain_adds(nxt_sp, b):
        for g in range(CHUNK_G):
            pltpu.make_async_copy(
                rows_v.at[b].at[pl.ds(g * GRP, GRP)],
                nxt_sp.at[idx_v.at[b].at[g].at[1]], asems[b]).wait()

    def scale(b):
        def scale_grp(k, _):
            for g in range(CHUNK_G):
                wv = plsc.bitcast(idx_v[b, g, 2, pl.ds(k * L, L)],
                                  jnp.float32)
                for e in range(L):
                    r = g * GRP + k * L + e
                    w = wv[e]
                    for s in range(DH // L):
                        sl = pl.ds(s * L, L)
                        rows_v[b, r, sl] = rows_v[b, r, sl] * w
            return 0
        lax.fori_loop(0, GRP // L, scale_grp, 0)

    def run_layer(in_hbm, nxt_sp):
        cps = [pltpu.async_copy(zb_v, nxt_sp.at[pl.ds(r0 + b * RB, RB)], gs0)
               for b in range(ROWS_PER_TILE // RB)]
        for cp in cps:
            cp.wait()
        plsc.subcore_barrier()

        pltpu.sync_copy(idx_hbm.at[pl.ds(g_base, CHUNK_G)], idx_v.at[0])
        issue_gathers(in_hbm, 0)
        load_idx_async(1, 1)

        def pipe(ii, _):
            for j in range(NBUF):
                c = ii * NBUF + j
                j1 = (j + 1) % NBUF
                j2 = (j + 2) % NBUF
                drain_gathers(in_hbm, j)

                @pl.when(c >= 2)
                def _():
                    drain_adds(nxt_sp, j2)

                @pl.when(c <= N_CHUNKS - 2)
                def _():
                    wait_idx(c + 1, j1)
                    issue_gathers(in_hbm, j1)

                @pl.when(c <= N_CHUNKS - 3)
                def _():
                    load_idx_async(c + 2, j2)
                scale(j)
                issue_adds(nxt_sp, j)
            return 0
        lax.fori_loop(0, N_CHUNKS // NBUF, pipe, 0)

        drain_adds(nxt_sp, (N_CHUNKS - 2) % NBUF)
        drain_adds(nxt_sp, (N_CHUNKS - 1) % NBUF)
        plsc.subcore_barrier()

    def spmem_to_hbm(sp, hbm_slice):
        srcs = [sp.at[pl.ds(r0, CH)], sp.at[pl.ds(r0 + CH, CH)],
                sp.at[pl.ds(r0 + 2 * CH, RB)]]
        dsts = [hbm_slice.at[pl.ds(r0, CH)], hbm_slice.at[pl.ds(r0 + CH, CH)],
                hbm_slice.at[pl.ds(r0 + 2 * CH, RB)]]
        stg = [rows_v.at[0], rows_v.at[1], rows_v.at[2].at[pl.ds(0, RB)]]
        rd = [pltpu.async_copy(srcs[i], stg[i], gsems[i]) for i in range(3)]
        wr = []
        for i in range(3):
            rd[i].wait()
            wr.append(pltpu.async_copy(stg[i], dsts[i], asems[i]))
        for cp in wr:
            cp.wait()

    for l in range(N_LAYERS):
        in_hbm = t0_hbm.at[cid] if l == 0 else tabs_hbm.at[l - 1, cid]
        run_layer(in_hbm, nxt_sp)
        spmem_to_hbm(nxt_sp, tabs_hbm.at[l, cid])
        plsc.subcore_barrier()

    def addrow(i, _):
        for cseg in range(DH // L):
            sl = pl.ds(cseg * L, L)
            zb_v[i, sl] = zb_v[i, sl] + rows_v[0, i, sl]
        return 0

    def mulrow(i, _):
        for cseg in range(DH // L):
            sl = pl.ds(cseg * L, L)
            zb_v[i, sl] = zb_v[i, sl] * 0.25
        return 0

    if True:
        for b in range(ROWS_PER_TILE // RB):
            rblk = pl.ds(r0 + b * RB, RB)
            pltpu.sync_copy(t0_hbm.at[cid].at[rblk], zb_v)
            for l in range(N_LAYERS):
                pltpu.sync_copy(tabs_hbm.at[l, cid].at[rblk],
                                rows_v.at[0].at[pl.ds(0, RB)])
                lax.fori_loop(0, RB, addrow, 0)
            lax.fori_loop(0, RB, mulrow, 0)
            pltpu.sync_copy(zb_v, out_hbm.at[cid].at[rblk])


@jax.jit
def kernel(users_emb, items_emb, edge_index, edge_weight):
    num_users = users_emb.shape[0]
    all_emb = jnp.concatenate(
        [users_emb, items_emb,
         jnp.zeros((N_PAD - N_NODES, D), jnp.float32)], axis=0)
    t0 = all_emb.reshape(N_PAD, NC, DH).transpose(1, 0, 2)

    dst = edge_index[0].astype(jnp.int32)
    src = edge_index[1].astype(jnp.int32)
    w = edge_weight

    npad = EDGES_PAD - N_EDGES
    pad_idx = (jnp.arange(npad, dtype=jnp.int32) % N_NODES)
    srcp = jnp.concatenate([src, pad_idx]).reshape(NGRP, GRP)
    dstp = jnp.concatenate([dst, pad_idx]).reshape(NGRP, GRP)
    wp = lax.bitcast_convert_type(
        jnp.concatenate([w, jnp.zeros((npad,), jnp.float32)]),
        jnp.int32).reshape(NGRP, GRP)
    idxp = jnp.stack([srcp, dstp, wp], axis=1)

    mesh = plsc.VectorSubcoreMesh(core_axis_name="c", subcore_axis_name="s")
    out, _tabs = pl.kernel(
        _body,
        out_type=(
            jax.ShapeDtypeStruct((NC, N_PAD, DH), jnp.float32),
            jax.ShapeDtypeStruct((N_LAYERS, NC, N_PAD, DH), jnp.float32),
        ),
        mesh=mesh,
        compiler_params=pltpu.CompilerParams(
            needs_layout_passes=False, use_tc_tiling_on_sc=False),
        scratch_types=[
            pltpu.VMEM((NBUF, CH, DH), jnp.float32),
            pltpu.VMEM((NBUF, CHUNK_G, 3, GRP), jnp.int32),
            pltpu.VMEM((RB, DH), jnp.float32),
            pltpu.VMEM_SHARED((N_PAD, DH), jnp.float32),
            pltpu.SemaphoreType.DMA,
            pltpu.SemaphoreType.DMA,
            pltpu.SemaphoreType.DMA,
            pltpu.SemaphoreType.DMA,
            pltpu.SemaphoreType.DMA,
            pltpu.SemaphoreType.DMA,
            pltpu.SemaphoreType.DMA,
            pltpu.SemaphoreType.DMA,
            pltpu.SemaphoreType.DMA,
            pltpu.SemaphoreType.DMA,
            pltpu.SemaphoreType.DMA,
            pltpu.SemaphoreType.DMA,
        ],
    )(t0, idxp)

    light = out.transpose(1, 0, 2).reshape(N_PAD, D)[:N_NODES]
    return (light[:num_users], light[num_users:])
